# baseline (speedup 1.0000x reference)
import jax
import jax.numpy as jnp
from jax import lax
from jax.experimental import pallas as pl
from jax.experimental.pallas import tpu as pltpu
from jax.experimental.pallas import tpu_sc as plsc

BATCH = 16384
NF = 26
D = 16
VOCAB = 100000
XW = 2 * NF
OUT_W = NF * D + NF

NC = 2
NS = 16
NW = NC * NS
B_PER_W = BATCH // NW
BC = 128
N_CHUNK = B_PER_W // BC
L = 16


def _body(xt_hbm, tab_hbm, out_hbm, xv, emb_v, stage_v, gsem, osem):
    wid = lax.axis_index("s") * NC + lax.axis_index("c")
    iota = lax.iota(jnp.int32, L)

    def chunk(c, carry):
        base = wid * B_PER_W + c * BC

        pltpu.sync_copy(xt_hbm.at[:, pl.ds(base, BC)], xv)

        gathers = [
            pltpu.async_copy(
                tab_hbm.at[f].at[xv.at[f]],
                emb_v.at[pl.ds(f * BC, BC), :],
                gsem,
            )
            for f in range(NF)
        ]

        for j in range(NF):
            for b0 in range(BC // L):
                stage_v[NF * D + j, pl.ds(b0 * L, L)] = (
                    xv[NF + j, pl.ds(b0 * L, L)].astype(jnp.float32)
                )

        for g in gathers:
            g.wait()

        def tr(f, carry2):
            for d in range(D):
                for b0 in range(BC // L):
                    vals = plsc.load_gather(
                        emb_v,
                        [f * BC + b0 * L + iota, jnp.full((L,), d, jnp.int32)],
                    )
                    stage_v[f * D + d, pl.ds(b0 * L, L)] = vals
            return carry2

        lax.fori_loop(0, NF, tr, 0)

        pltpu.async_copy(stage_v, out_hbm.at[:, pl.ds(base, BC)], osem).wait()
        return carry

    lax.fori_loop(0, N_CHUNK, chunk, 0)


@jax.jit
def _emb_lookup(xt, tab3):
    run = pl.kernel(
        _body,
        out_type=jax.ShapeDtypeStruct((OUT_W, BATCH), jnp.float32),
        mesh=plsc.VectorSubcoreMesh(
            core_axis_name="c", subcore_axis_name="s", num_cores=NC,
            num_subcores=NS,
        ),
        scratch_types=[
            pltpu.VMEM((XW, BC), jnp.int32),
            pltpu.VMEM((NF * BC, D), jnp.float32),
            pltpu.VMEM((OUT_W, BC), jnp.float32),
            pltpu.SemaphoreType.DMA,
            pltpu.SemaphoreType.DMA,
        ],
        compiler_params=pltpu.CompilerParams(
            use_tc_tiling_on_sc=False, needs_layout_passes=False
        ),
    )
    return run(xt, tab3)


def kernel(x, tables):
    return _emb_lookup(x.T, tables).T
